# baseline (speedup 1.0000x reference)
import functools

import jax
import jax.numpy as jnp
from jax import lax
from jax.experimental import pallas as pl
from jax.experimental.pallas import tpu as pltpu
from jax.experimental.pallas import tpu_sc as plsc

N = 4096
D = 1024
E = 8
F = 512
FS = 1024
SCALE = 2.5

BMA = 1024
BMG = 256
LOG_BMG = 8
NP = 2 * N + E * BMG
NB = NP // BMG
NBPAD = 64


def _router_body(x_ref, wr_ref, route_ref, topw_ref):
    x32 = x_ref[...]
    logits = jnp.dot(x32, wr_ref[...], preferred_element_type=jnp.float32)
    m = jnp.max(logits, axis=-1, keepdims=True)
    p = jnp.exp(logits - m)
    scores = p / jnp.sum(p, axis=-1, keepdims=True)
    lane = lax.broadcasted_iota(jnp.int32, scores.shape, 1)
    v1 = jnp.max(scores, axis=-1, keepdims=True)
    i1 = jnp.min(jnp.where(scores == v1, lane, E), axis=-1, keepdims=True)
    masked = jnp.where(lane == i1, -1e30, scores)
    v2 = jnp.max(masked, axis=-1, keepdims=True)
    i2 = jnp.min(jnp.where(masked == v2, lane, E), axis=-1, keepdims=True)
    s = v1 + v2
    route_ref[...] = i1 + E * i2
    topw_ref[...] = jnp.concatenate([v1 / s, v2 / s], axis=1) * SCALE


def _router(x2, w_router):
    rb = N // BMA
    return pl.pallas_call(
        _router_body,
        grid=(rb,),
        in_specs=[
            pl.BlockSpec((BMA, D), lambda i: (i, 0)),
            pl.BlockSpec((D, E), lambda i: (0, 0)),
        ],
        out_specs=[
            pl.BlockSpec((BMA, 1), lambda i: (i, 0)),
            pl.BlockSpec((BMA, 2), lambda i: (i, 0)),
        ],
        out_shape=[
            jax.ShapeDtypeStruct((N, 1), jnp.int32),
            jax.ShapeDtypeStruct((N, 2), jnp.float32),
        ],
        compiler_params=pltpu.CompilerParams(
            dimension_semantics=("arbitrary",),
        ),
    )(x2, w_router)


def _shared_body(x_ref, sg_ref, su_ref, sd_ref, shared_ref):
    xb = x_ref[...].astype(jnp.bfloat16)
    g = jnp.dot(xb, sg_ref[...], preferred_element_type=jnp.float32)
    u = jnp.dot(xb, su_ref[...], preferred_element_type=jnp.float32)
    h = (g * jax.nn.sigmoid(g) * u).astype(jnp.bfloat16)
    shared_ref[...] = jnp.dot(h, sd_ref[...], preferred_element_type=jnp.float32)


def _shared(x2, sg, su, sd):
    rb = N // BMA
    return pl.pallas_call(
        _shared_body,
        grid=(rb,),
        in_specs=[
            pl.BlockSpec((BMA, D), lambda i: (i, 0)),
            pl.BlockSpec((D, FS), lambda i: (0, 0)),
            pl.BlockSpec((D, FS), lambda i: (0, 0)),
            pl.BlockSpec((FS, D), lambda i: (0, 0)),
        ],
        out_specs=pl.BlockSpec((BMA, D), lambda i: (i, 0)),
        out_shape=jax.ShapeDtypeStruct((N, D), jnp.float32),
        compiler_params=pltpu.CompilerParams(
            dimension_semantics=("arbitrary",),
        ),
    )(x2, sg, su, sd)


_TPW = N // 16


def _stage(x, buf_v):
    z = jnp.zeros((16,), jnp.int32)
    buf_v[pl.ds(0, 16)] = z
    buf_v[pl.ds(16, 16)] = x
    buf_v[pl.ds(32, 16)] = z


def _prefix16(x, buf_v):
    for k in (1, 2, 4, 8):
        _stage(x, buf_v)
        x = x + buf_v[pl.ds(16 - k, 16)]
    return x


def _splat_last(x, buf_v):
    for k in (1, 2, 4, 8):
        _stage(x, buf_v)
        x = jnp.maximum(x, buf_v[pl.ds(16 + k, 16)])
    return x


def _splat_lane(x, e, buf_v):
    if e:
        _stage(x, buf_v)
        y = buf_v[pl.ds(16 + e, 16)]
    else:
        y = x
    y = y * _mask01(lax.iota(jnp.int32, 16) == 0)
    for k in (1, 2, 4, 8):
        _stage(y, buf_v)
        y = jnp.maximum(y, buf_v[pl.ds(16 - k, 16)])
    return y


def _mask01(pred):
    return jnp.where(pred, jnp.full((16,), 1, jnp.int32),
                     jnp.zeros((16,), jnp.int32))


def _plan1_body(route_hbm, counts_hbm, ranks_hbm,
                pk_v, eid_v, rank_v, cnt_v, buf_v):
    wid = lax.axis_index("s")
    t0 = wid * _TPW
    pltpu.sync_copy(route_hbm.at[pl.ds(t0, _TPW)], pk_v)
    nv = _TPW // 16
    for v in range(nv):
        pk = pk_v[pl.ds(v * 16, 16)]
        eid_v[pl.ds(v * 16, 16)] = pk & 7
        eid_v[pl.ds(_TPW + v * 16, 16)] = pk >> 3
    zero16 = jnp.zeros((16,), jnp.int32)
    cnt_v[...] = zero16
    lane16 = lax.iota(jnp.int32, 16)
    for e in range(E):
        def body(v, run, e=e):
            x = eid_v[pl.ds(v * 16, 16)]
            mi = _mask01(x == e)
            s = _prefix16(mi, buf_v)
            old = rank_v[pl.ds(v * 16, 16)]
            rank_v[pl.ds(v * 16, 16)] = old + mi * (run + s - 1 - old)
            return run + _splat_last(s, buf_v)
        run_f = lax.fori_loop(0, 2 * nv, body, jnp.zeros((16,), jnp.int32))
        cur = cnt_v[...]
        cnt_v[...] = cur + _mask01(lane16 == e) * (run_f - cur)
    pltpu.sync_copy(cnt_v, counts_hbm.at[wid])
    pltpu.sync_copy(rank_v.at[pl.ds(0, _TPW)],
                    ranks_hbm.at[0, pl.ds(t0, _TPW)])
    pltpu.sync_copy(rank_v.at[pl.ds(_TPW, _TPW)],
                    ranks_hbm.at[1, pl.ds(t0, _TPW)])


def _plan2_body(route_hbm, counts_hbm, ranks_hbm, slots_hbm, blk_hbm,
                pk_v, eid_v, rank_v, dest_v, cnt_v, base_v, blk_v,
                allc_v, buf_v, basesp_v, offsp_v):
    wid = lax.axis_index("s")
    t0 = wid * _TPW
    pltpu.sync_copy(route_hbm.at[pl.ds(t0, _TPW)], pk_v)
    nv = _TPW // 16
    for v in range(nv):
        pk = pk_v[pl.ds(v * 16, 16)]
        eid_v[pl.ds(v * 16, 16)] = pk & 7
        eid_v[pl.ds(_TPW + v * 16, 16)] = pk >> 3
    pltpu.sync_copy(ranks_hbm.at[0, pl.ds(t0, _TPW)],
                    rank_v.at[pl.ds(0, _TPW)])
    pltpu.sync_copy(ranks_hbm.at[1, pl.ds(t0, _TPW)],
                    rank_v.at[pl.ds(_TPW, _TPW)])
    pltpu.sync_copy(counts_hbm, allc_v)
    zero16 = jnp.zeros((16,), jnp.int32)
    lane16 = lax.iota(jnp.int32, 16)
    widv = zero16 + wid
    total = jnp.zeros((16,), jnp.int32)
    pref = jnp.zeros((16,), jnp.int32)
    for w in range(16):
        row = allc_v[w, :]
        total = total + row
        pref = pref + row * _mask01(jnp.full((16,), w, jnp.int32) < widv)
    aligned = ((total + (BMG - 1)) >> LOG_BMG) << LOG_BMG
    offi = _prefix16(aligned, buf_v)
    base = (offi - aligned) + pref
    base_v[...] = base
    for e in range(E):
        basesp_v[e, :] = _splat_lane(base, e, buf_v)
        offsp_v[e, :] = _splat_lane(offi, e, buf_v)
    def bodyc(v, _):
        sl = pl.ds(v * 16, 16)
        ev = eid_v[sl]
        d = rank_v[sl]
        for e in range(E):
            d = d + basesp_v[e, :] * _mask01(ev == e)
        dest_v[sl] = d
        return 0
    lax.fori_loop(0, 2 * nv, bodyc, 0)
    pltpu.sync_copy(dest_v.at[pl.ds(0, _TPW)], slots_hbm.at[0, pl.ds(t0, _TPW)])
    pltpu.sync_copy(dest_v.at[pl.ds(_TPW, _TPW)], slots_hbm.at[1, pl.ds(t0, _TPW)])

    @pl.when(wid == 0)
    def _():
        for v in range(NBPAD // 16):
            pos = (lax.iota(jnp.int32, 16) + v * 16) * BMG
            cnt = jnp.zeros((16,), jnp.int32)
            for e in range(E):
                cnt = cnt + _mask01(pos >= offsp_v[e, :])
            blk_v[pl.ds(v * 16, 16)] = jnp.minimum(cnt, E - 1)
        pltpu.sync_copy(blk_v, blk_hbm)


@functools.cache
def _make_plan1():
    mesh = plsc.VectorSubcoreMesh(
        core_axis_name="c", subcore_axis_name="s", num_cores=1,
        num_subcores=16)

    @functools.partial(
        pl.kernel,
        out_type=[
            jax.ShapeDtypeStruct((16, 16), jnp.int32),
            jax.ShapeDtypeStruct((2, N), jnp.int32),
        ],
        mesh=mesh,
        scratch_types=[
            pltpu.VMEM((_TPW,), jnp.int32),
            pltpu.VMEM((2 * _TPW,), jnp.int32),
            pltpu.VMEM((2 * _TPW,), jnp.int32),
            pltpu.VMEM((16,), jnp.int32),
            pltpu.VMEM((48,), jnp.int32),
        ],
    )
    def _plan1(route_hbm, counts_hbm, ranks_hbm, *rest):
        _plan1_body(route_hbm, counts_hbm, ranks_hbm, *rest)

    return _plan1


@functools.cache
def _make_plan2():
    mesh = plsc.VectorSubcoreMesh(
        core_axis_name="c", subcore_axis_name="s", num_cores=1,
        num_subcores=16)

    @functools.partial(
        pl.kernel,
        out_type=[
            jax.ShapeDtypeStruct((2, N), jnp.int32),
            jax.ShapeDtypeStruct((NBPAD,), jnp.int32),
        ],
        mesh=mesh,
        scratch_types=[
            pltpu.VMEM((_TPW,), jnp.int32),
            pltpu.VMEM((2 * _TPW,), jnp.int32),
            pltpu.VMEM((2 * _TPW,), jnp.int32),
            pltpu.VMEM((2 * _TPW,), jnp.int32),
            pltpu.VMEM((16,), jnp.int32),
            pltpu.VMEM((16,), jnp.int32),
            pltpu.VMEM((NBPAD,), jnp.int32),
            pltpu.VMEM((16, 16), jnp.int32),
            pltpu.VMEM((48,), jnp.int32),
            pltpu.VMEM((E, 16), jnp.int32),
            pltpu.VMEM((E, 16), jnp.int32),
        ],
    )
    def _plan2(route_hbm, counts_hbm, ranks_hbm, slots_hbm, blk_hbm, *rest):
        _plan2_body(route_hbm, counts_hbm, ranks_hbm, slots_hbm, blk_hbm,
                    *rest)

    return _plan2


def _plan_full(route_flat):
    counts, ranks = _make_plan1()(route_flat)
    return _make_plan2()(route_flat, counts, ranks)


_TPD = N // 32
_CHUNK = 32


def _disp_mesh():
    return plsc.VectorSubcoreMesh(
        core_axis_name="c", subcore_axis_name="s", num_cores=2,
        num_subcores=16)


@functools.cache
def _make_dispatch():
    @functools.partial(
        pl.kernel,
        out_type=jax.ShapeDtypeStruct((NP, D), jnp.float32),
        mesh=_disp_mesh(),
        scratch_types=[
            pltpu.VMEM((_CHUNK, D), jnp.float32),
            pltpu.VMEM((2, _CHUNK), jnp.int32),
            pltpu.SemaphoreType.DMA,
        ],
    )
    def _dispatch(x_hbm, slots_hbm, xs_hbm, rows_v, idx_v, sem):
        wid = lax.axis_index("s") * 2 + lax.axis_index("c")
        for c in range(_TPD // _CHUNK):
            t0 = wid * _TPD + c * _CHUNK
            pltpu.sync_copy(x_hbm.at[pl.ds(t0, _CHUNK), :], rows_v)
            pltpu.sync_copy(slots_hbm.at[0, pl.ds(t0, _CHUNK)], idx_v.at[0])
            pltpu.sync_copy(slots_hbm.at[1, pl.ds(t0, _CHUNK)], idx_v.at[1])
            cp0 = pltpu.async_copy(rows_v, xs_hbm.at[idx_v.at[0]], sem)
            cp1 = pltpu.async_copy(rows_v, xs_hbm.at[idx_v.at[1]], sem)
            cp0.wait()
            cp1.wait()

    return _dispatch


def _gemm_body(blk_ref, xs_ref, gw_ref, uw_ref, dw_ref, ys_ref):
    xb = xs_ref[...].astype(jnp.bfloat16)
    g = jnp.dot(xb, gw_ref[0], preferred_element_type=jnp.float32)
    u = jnp.dot(xb, uw_ref[0], preferred_element_type=jnp.float32)
    h = (g * jax.nn.sigmoid(g) * u).astype(jnp.bfloat16)
    ys_ref[...] = jnp.dot(h, dw_ref[0], preferred_element_type=jnp.float32)


def _grouped_gemm(blk, xs, gw, uw, dw):
    return pl.pallas_call(
        _gemm_body,
        grid_spec=pltpu.PrefetchScalarGridSpec(
            num_scalar_prefetch=1,
            grid=(NB,),
            in_specs=[
                pl.BlockSpec((BMG, D), lambda i, blk_ref: (i, 0)),
                pl.BlockSpec((1, D, F), lambda i, blk_ref: (blk_ref[i], 0, 0)),
                pl.BlockSpec((1, D, F), lambda i, blk_ref: (blk_ref[i], 0, 0)),
                pl.BlockSpec((1, F, D), lambda i, blk_ref: (blk_ref[i], 0, 0)),
            ],
            out_specs=pl.BlockSpec((BMG, D), lambda i, blk_ref: (i, 0)),
        ),
        out_shape=jax.ShapeDtypeStruct((NP, D), jnp.float32),
        compiler_params=pltpu.CompilerParams(
            dimension_semantics=("arbitrary",),
        ),
    )(blk, xs, gw, uw, dw)


@functools.cache
def _make_combine():
    @functools.partial(
        pl.kernel,
        out_type=jax.ShapeDtypeStruct((N, D), jnp.float32),
        mesh=_disp_mesh(),
        scratch_types=[
            pltpu.VMEM((_CHUNK, D), jnp.float32),
            pltpu.VMEM((_CHUNK, D), jnp.float32),
            pltpu.VMEM((_CHUNK, D), jnp.float32),
            pltpu.VMEM((_CHUNK,), jnp.int32),
            pltpu.VMEM((_CHUNK,), jnp.int32),
            pltpu.VMEM((2 * _CHUNK,), jnp.float32),
            pltpu.SemaphoreType.DMA,
        ],
    )
    def _combine(shared_hbm, ys_hbm, slots_hbm, topw_hbm, out_hbm,
                 acc_v, y0_v, y1_v, i0_v, i1_v, w_v, sem):
        wid = lax.axis_index("s") * 2 + lax.axis_index("c")
        for c in range(_TPD // _CHUNK):
            t0 = wid * _TPD + c * _CHUNK
            pltpu.sync_copy(slots_hbm.at[0, pl.ds(t0, _CHUNK)], i0_v)
            pltpu.sync_copy(slots_hbm.at[1, pl.ds(t0, _CHUNK)], i1_v)
            pltpu.sync_copy(topw_hbm.at[pl.ds(2 * t0, 2 * _CHUNK)], w_v)
            g0 = pltpu.async_copy(ys_hbm.at[i0_v], y0_v, sem)
            g1 = pltpu.async_copy(ys_hbm.at[i1_v], y1_v, sem)
            pltpu.sync_copy(shared_hbm.at[pl.ds(t0, _CHUNK), :], acc_v)
            g0.wait()
            g1.wait()
            zero16f = jnp.zeros((16,), jnp.float32)
            for g in range(_CHUNK // 8):
                wg = w_v[pl.ds(g * 16, 16)]
                for j in range(8):
                    r = g * 8 + j
                    w0 = zero16f + wg[2 * j]
                    w1 = zero16f + wg[2 * j + 1]
                    def bodyr(v, _, r=r, w0=w0, w1=w1):
                        sl = pl.ds(v * 16, 16)
                        acc_v[r, sl] = (acc_v[r, sl] + w0 * y0_v[r, sl]
                                        + w1 * y1_v[r, sl])
                        return 0
                    lax.fori_loop(0, D // 16, bodyr, 0)
            pltpu.sync_copy(acc_v, out_hbm.at[pl.ds(t0, _CHUNK), :])

    return _combine


def kernel(hidden_states, w_router, gate_w, up_w, down_w,
           shared_gate_w, shared_up_w, shared_down_w):
    shape = hidden_states.shape
    x2 = hidden_states.reshape(N, D)
    gw = gate_w.astype(jnp.bfloat16)
    uw = up_w.astype(jnp.bfloat16)
    dw = down_w.astype(jnp.bfloat16)
    sg = shared_gate_w.astype(jnp.bfloat16)
    su = shared_up_w.astype(jnp.bfloat16)
    sd = shared_down_w.astype(jnp.bfloat16)

    route, topw = _router(x2, w_router)
    slots, blk = _plan_full(route.reshape(N))
    xs = _make_dispatch()(x2, slots)
    shared_out = _shared(x2, sg, su, sd)
    ys = _grouped_gemm(blk, xs, gw, uw, dw)
    out = _make_combine()(shared_out, ys, slots, topw.reshape(2 * N))
    return out.reshape(shape)


# speedup vs baseline: 1.0153x; 1.0153x over previous
import functools

import jax
import jax.numpy as jnp
from jax import lax
from jax.experimental import pallas as pl
from jax.experimental.pallas import tpu as pltpu
from jax.experimental.pallas import tpu_sc as plsc

N = 4096
D = 1024
E = 8
F = 512
FS = 1024
SCALE = 2.5

BMA = 1024
BMG = 256
LOG_BMG = 8
NP = 2 * N + E * BMG
NB = NP // BMG
NBPAD = 64


def _router_body(x_ref, wr_ref, route_ref, topw_ref):
    x32 = x_ref[...]
    logits = jnp.dot(x32, wr_ref[...], preferred_element_type=jnp.float32)
    m = jnp.max(logits, axis=-1, keepdims=True)
    p = jnp.exp(logits - m)
    scores = p / jnp.sum(p, axis=-1, keepdims=True)
    lane = lax.broadcasted_iota(jnp.int32, scores.shape, 1)
    v1 = jnp.max(scores, axis=-1, keepdims=True)
    i1 = jnp.min(jnp.where(scores == v1, lane, E), axis=-1, keepdims=True)
    masked = jnp.where(lane == i1, -1e30, scores)
    v2 = jnp.max(masked, axis=-1, keepdims=True)
    i2 = jnp.min(jnp.where(masked == v2, lane, E), axis=-1, keepdims=True)
    s = v1 + v2
    route_ref[...] = i1 + E * i2
    topw_ref[...] = jnp.concatenate([v1 / s, v2 / s], axis=1) * SCALE


def _router(x2, w_router):
    rb = N // BMA
    return pl.pallas_call(
        _router_body,
        grid=(rb,),
        in_specs=[
            pl.BlockSpec((BMA, D), lambda i: (i, 0)),
            pl.BlockSpec((D, E), lambda i: (0, 0)),
        ],
        out_specs=[
            pl.BlockSpec((BMA, 1), lambda i: (i, 0)),
            pl.BlockSpec((BMA, 2), lambda i: (i, 0)),
        ],
        out_shape=[
            jax.ShapeDtypeStruct((N, 1), jnp.int32),
            jax.ShapeDtypeStruct((N, 2), jnp.float32),
        ],
        compiler_params=pltpu.CompilerParams(
            dimension_semantics=("arbitrary",),
        ),
    )(x2, w_router)


def _shared_body(x_ref, sg_ref, su_ref, sd_ref, shared_ref):
    xb = x_ref[...].astype(jnp.bfloat16)
    g = jnp.dot(xb, sg_ref[...], preferred_element_type=jnp.float32)
    u = jnp.dot(xb, su_ref[...], preferred_element_type=jnp.float32)
    h = (g * jax.nn.sigmoid(g) * u).astype(jnp.bfloat16)
    shared_ref[...] = jnp.dot(h, sd_ref[...], preferred_element_type=jnp.float32)


def _shared(x2, sg, su, sd):
    rb = N // BMA
    return pl.pallas_call(
        _shared_body,
        grid=(rb,),
        in_specs=[
            pl.BlockSpec((BMA, D), lambda i: (i, 0)),
            pl.BlockSpec((D, FS), lambda i: (0, 0)),
            pl.BlockSpec((D, FS), lambda i: (0, 0)),
            pl.BlockSpec((FS, D), lambda i: (0, 0)),
        ],
        out_specs=pl.BlockSpec((BMA, D), lambda i: (i, 0)),
        out_shape=jax.ShapeDtypeStruct((N, D), jnp.float32),
        compiler_params=pltpu.CompilerParams(
            dimension_semantics=("arbitrary",),
        ),
    )(x2, sg, su, sd)


_TPW = N // 16


def _stage(x, buf_v):
    z = jnp.zeros((16,), jnp.int32)
    buf_v[pl.ds(0, 16)] = z
    buf_v[pl.ds(16, 16)] = x
    buf_v[pl.ds(32, 16)] = z


def _prefix16(x, buf_v):
    for k in (1, 2, 4, 8):
        _stage(x, buf_v)
        x = x + buf_v[pl.ds(16 - k, 16)]
    return x


def _splat_last(x, buf_v):
    for k in (1, 2, 4, 8):
        _stage(x, buf_v)
        x = jnp.maximum(x, buf_v[pl.ds(16 + k, 16)])
    return x


def _splat_lane(x, e, buf_v):
    if e:
        _stage(x, buf_v)
        y = buf_v[pl.ds(16 + e, 16)]
    else:
        y = x
    y = y * _mask01(lax.iota(jnp.int32, 16) == 0)
    for k in (1, 2, 4, 8):
        _stage(y, buf_v)
        y = jnp.maximum(y, buf_v[pl.ds(16 - k, 16)])
    return y


def _mask01(pred):
    return jnp.where(pred, jnp.full((16,), 1, jnp.int32),
                     jnp.zeros((16,), jnp.int32))


def _plan1_body(route_hbm, counts_hbm, ranks_hbm,
                pk_v, eid_v, rank_v, cnt_v, buf_v):
    wid = lax.axis_index("s")
    t0 = wid * _TPW
    pltpu.sync_copy(route_hbm.at[pl.ds(t0, _TPW)], pk_v)
    nv = _TPW // 16
    for v in range(nv):
        pk = pk_v[pl.ds(v * 16, 16)]
        eid_v[pl.ds(v * 16, 16)] = pk & 7
        eid_v[pl.ds(_TPW + v * 16, 16)] = pk >> 3
    zero16 = jnp.zeros((16,), jnp.int32)
    cnt_v[...] = zero16
    lane16 = lax.iota(jnp.int32, 16)
    for e in range(E):
        def body(v, run, e=e):
            x = eid_v[pl.ds(v * 16, 16)]
            mi = _mask01(x == e)
            s = _prefix16(mi, buf_v)
            old = rank_v[pl.ds(v * 16, 16)]
            rank_v[pl.ds(v * 16, 16)] = old + mi * (run + s - 1 - old)
            return run + _splat_last(s, buf_v)
        run_f = lax.fori_loop(0, 2 * nv, body, jnp.zeros((16,), jnp.int32))
        cur = cnt_v[...]
        cnt_v[...] = cur + _mask01(lane16 == e) * (run_f - cur)
    pltpu.sync_copy(cnt_v, counts_hbm.at[wid])
    pltpu.sync_copy(rank_v.at[pl.ds(0, _TPW)],
                    ranks_hbm.at[0, pl.ds(t0, _TPW)])
    pltpu.sync_copy(rank_v.at[pl.ds(_TPW, _TPW)],
                    ranks_hbm.at[1, pl.ds(t0, _TPW)])


def _plan2_body(route_hbm, counts_hbm, ranks_hbm, slots_hbm, blk_hbm,
                pk_v, eid_v, rank_v, dest_v, cnt_v, base_v, blk_v,
                allc_v, buf_v, basesp_v, offsp_v):
    wid = lax.axis_index("s")
    t0 = wid * _TPW
    pltpu.sync_copy(route_hbm.at[pl.ds(t0, _TPW)], pk_v)
    nv = _TPW // 16
    for v in range(nv):
        pk = pk_v[pl.ds(v * 16, 16)]
        eid_v[pl.ds(v * 16, 16)] = pk & 7
        eid_v[pl.ds(_TPW + v * 16, 16)] = pk >> 3
    pltpu.sync_copy(ranks_hbm.at[0, pl.ds(t0, _TPW)],
                    rank_v.at[pl.ds(0, _TPW)])
    pltpu.sync_copy(ranks_hbm.at[1, pl.ds(t0, _TPW)],
                    rank_v.at[pl.ds(_TPW, _TPW)])
    pltpu.sync_copy(counts_hbm, allc_v)
    zero16 = jnp.zeros((16,), jnp.int32)
    lane16 = lax.iota(jnp.int32, 16)
    widv = zero16 + wid
    total = jnp.zeros((16,), jnp.int32)
    pref = jnp.zeros((16,), jnp.int32)
    for w in range(16):
        row = allc_v[w, :]
        total = total + row
        pref = pref + row * _mask01(jnp.full((16,), w, jnp.int32) < widv)
    aligned = ((total + (BMG - 1)) >> LOG_BMG) << LOG_BMG
    offi = _prefix16(aligned, buf_v)
    base = (offi - aligned) + pref
    base_v[...] = base
    for e in range(E):
        basesp_v[e, :] = _splat_lane(base, e, buf_v)
        offsp_v[e, :] = _splat_lane(offi, e, buf_v)
    def bodyc(v, _):
        sl = pl.ds(v * 16, 16)
        ev = eid_v[sl]
        d = rank_v[sl]
        for e in range(E):
            d = d + basesp_v[e, :] * _mask01(ev == e)
        dest_v[sl] = d
        return 0
    lax.fori_loop(0, 2 * nv, bodyc, 0)
    pltpu.sync_copy(dest_v.at[pl.ds(0, _TPW)], slots_hbm.at[0, pl.ds(t0, _TPW)])
    pltpu.sync_copy(dest_v.at[pl.ds(_TPW, _TPW)], slots_hbm.at[1, pl.ds(t0, _TPW)])

    @pl.when(wid == 0)
    def _():
        for v in range(NBPAD // 16):
            pos = (lax.iota(jnp.int32, 16) + v * 16) * BMG
            cnt = jnp.zeros((16,), jnp.int32)
            for e in range(E):
                cnt = cnt + _mask01(pos >= offsp_v[e, :])
            blk_v[pl.ds(v * 16, 16)] = jnp.minimum(cnt, E - 1)
        pltpu.sync_copy(blk_v, blk_hbm)


@functools.cache
def _make_plan1():
    mesh = plsc.VectorSubcoreMesh(
        core_axis_name="c", subcore_axis_name="s", num_cores=1,
        num_subcores=16)

    @functools.partial(
        pl.kernel,
        out_type=[
            jax.ShapeDtypeStruct((16, 16), jnp.int32),
            jax.ShapeDtypeStruct((2, N), jnp.int32),
        ],
        mesh=mesh,
        scratch_types=[
            pltpu.VMEM((_TPW,), jnp.int32),
            pltpu.VMEM((2 * _TPW,), jnp.int32),
            pltpu.VMEM((2 * _TPW,), jnp.int32),
            pltpu.VMEM((16,), jnp.int32),
            pltpu.VMEM((48,), jnp.int32),
        ],
    )
    def _plan1(route_hbm, counts_hbm, ranks_hbm, *rest):
        _plan1_body(route_hbm, counts_hbm, ranks_hbm, *rest)

    return _plan1


@functools.cache
def _make_plan2():
    mesh = plsc.VectorSubcoreMesh(
        core_axis_name="c", subcore_axis_name="s", num_cores=1,
        num_subcores=16)

    @functools.partial(
        pl.kernel,
        out_type=[
            jax.ShapeDtypeStruct((2, N), jnp.int32),
            jax.ShapeDtypeStruct((NBPAD,), jnp.int32),
        ],
        mesh=mesh,
        scratch_types=[
            pltpu.VMEM((_TPW,), jnp.int32),
            pltpu.VMEM((2 * _TPW,), jnp.int32),
            pltpu.VMEM((2 * _TPW,), jnp.int32),
            pltpu.VMEM((2 * _TPW,), jnp.int32),
            pltpu.VMEM((16,), jnp.int32),
            pltpu.VMEM((16,), jnp.int32),
            pltpu.VMEM((NBPAD,), jnp.int32),
            pltpu.VMEM((16, 16), jnp.int32),
            pltpu.VMEM((48,), jnp.int32),
            pltpu.VMEM((E, 16), jnp.int32),
            pltpu.VMEM((E, 16), jnp.int32),
        ],
    )
    def _plan2(route_hbm, counts_hbm, ranks_hbm, slots_hbm, blk_hbm, *rest):
        _plan2_body(route_hbm, counts_hbm, ranks_hbm, slots_hbm, blk_hbm,
                    *rest)

    return _plan2


def _plan_full(route_flat):
    counts, ranks = _make_plan1()(route_flat)
    return _make_plan2()(route_flat, counts, ranks)


_TPD = N // 32
_CHUNK = 32


def _disp_mesh():
    return plsc.VectorSubcoreMesh(
        core_axis_name="c", subcore_axis_name="s", num_cores=2,
        num_subcores=16)


@functools.cache
def _make_dispatch():
    @functools.partial(
        pl.kernel,
        out_type=jax.ShapeDtypeStruct((NP, D), jnp.float32),
        mesh=_disp_mesh(),
        scratch_types=[
            pltpu.VMEM((_CHUNK, D), jnp.float32),
            pltpu.VMEM((2, _CHUNK), jnp.int32),
            pltpu.SemaphoreType.DMA,
        ],
    )
    def _dispatch(x_hbm, slots_hbm, xs_hbm, rows_v, idx_v, sem):
        wid = lax.axis_index("s") * 2 + lax.axis_index("c")
        for c in range(_TPD // _CHUNK):
            t0 = wid * _TPD + c * _CHUNK
            pltpu.sync_copy(x_hbm.at[pl.ds(t0, _CHUNK), :], rows_v)
            pltpu.sync_copy(slots_hbm.at[0, pl.ds(t0, _CHUNK)], idx_v.at[0])
            pltpu.sync_copy(slots_hbm.at[1, pl.ds(t0, _CHUNK)], idx_v.at[1])
            cp0 = pltpu.async_copy(rows_v, xs_hbm.at[idx_v.at[0]], sem)
            cp1 = pltpu.async_copy(rows_v, xs_hbm.at[idx_v.at[1]], sem)
            cp0.wait()
            cp1.wait()

    return _dispatch


def _gemm_body(blk_ref, xs_ref, gw_ref, uw_ref, dw_ref, ys_ref):
    xb = xs_ref[...].astype(jnp.bfloat16)
    g = jnp.dot(xb, gw_ref[0], preferred_element_type=jnp.float32)
    u = jnp.dot(xb, uw_ref[0], preferred_element_type=jnp.float32)
    h = (g * jax.nn.sigmoid(g) * u).astype(jnp.bfloat16)
    ys_ref[...] = jnp.dot(h, dw_ref[0], preferred_element_type=jnp.float32)


def _grouped_gemm(blk, xs, gw, uw, dw):
    return pl.pallas_call(
        _gemm_body,
        grid_spec=pltpu.PrefetchScalarGridSpec(
            num_scalar_prefetch=1,
            grid=(NB,),
            in_specs=[
                pl.BlockSpec((BMG, D), lambda i, blk_ref: (i, 0)),
                pl.BlockSpec((1, D, F), lambda i, blk_ref: (blk_ref[i], 0, 0)),
                pl.BlockSpec((1, D, F), lambda i, blk_ref: (blk_ref[i], 0, 0)),
                pl.BlockSpec((1, F, D), lambda i, blk_ref: (blk_ref[i], 0, 0)),
            ],
            out_specs=pl.BlockSpec((BMG, D), lambda i, blk_ref: (i, 0)),
        ),
        out_shape=jax.ShapeDtypeStruct((NP, D), jnp.float32),
        compiler_params=pltpu.CompilerParams(
            dimension_semantics=("arbitrary",),
        ),
    )(blk, xs, gw, uw, dw)


_CCH = 16


@functools.cache
def _make_combine():
    nc = _TPD // _CCH

    @functools.partial(
        pl.kernel,
        out_type=jax.ShapeDtypeStruct((N, D), jnp.float32),
        mesh=_disp_mesh(),
        scratch_types=[
            pltpu.VMEM((_CCH, D), jnp.float32),
            pltpu.VMEM((_CCH, D), jnp.float32),
            pltpu.VMEM((_CCH, D), jnp.float32),
            pltpu.VMEM((_CCH, D), jnp.float32),
            pltpu.VMEM((_CCH, D), jnp.float32),
            pltpu.VMEM((2, _CCH), jnp.int32),
            pltpu.VMEM((2, _CCH), jnp.int32),
            pltpu.VMEM((2 * _CCH,), jnp.float32),
            pltpu.SemaphoreType.DMA,
        ],
    )
    def _combine(shared_hbm, ys_hbm, slots_hbm, topw_hbm, out_hbm,
                 acc_v, y0a, y0b, y1a, y1b, idxa, idxb, w_v, sem):
        wid = lax.axis_index("s") * 2 + lax.axis_index("c")
        y0 = (y0a, y0b)
        y1 = (y1a, y1b)
        idx = (idxa, idxb)

        def start_gather(c):
            b = c & 1
            t0 = wid * _TPD + c * _CCH
            pltpu.sync_copy(slots_hbm.at[0, pl.ds(t0, _CCH)], idx[b].at[0])
            pltpu.sync_copy(slots_hbm.at[1, pl.ds(t0, _CCH)], idx[b].at[1])
            return (
                pltpu.async_copy(ys_hbm.at[idx[b].at[0]], y0[b], sem),
                pltpu.async_copy(ys_hbm.at[idx[b].at[1]], y1[b], sem),
            )

        g = {0: start_gather(0)}
        for c in range(nc):
            b = c & 1
            t0 = wid * _TPD + c * _CCH
            pltpu.sync_copy(topw_hbm.at[pl.ds(2 * t0, 2 * _CCH)], w_v)
            pltpu.sync_copy(shared_hbm.at[pl.ds(t0, _CCH), :], acc_v)
            if c + 1 < nc:
                g[c + 1] = start_gather(c + 1)
            for d in g[c]:
                d.wait()
            zf = jnp.zeros((16,), jnp.float32)
            for gi in range(_CCH // 8):
                wg = w_v[pl.ds(gi * 16, 16)]
                for j in range(8):
                    r = gi * 8 + j
                    w0 = zf + wg[2 * j]
                    w1 = zf + wg[2 * j + 1]
                    def bodyr(v, _, b=b, r=r, w0=w0, w1=w1):
                        sl = pl.ds(v * 16, 16)
                        acc_v[r, sl] = (acc_v[r, sl] + w0 * y0[b][r, sl]
                                        + w1 * y1[b][r, sl])
                        return 0
                    lax.fori_loop(0, D // 16, bodyr, 0)
            pltpu.sync_copy(acc_v, out_hbm.at[pl.ds(t0, _CCH), :])

    return _combine


def kernel(hidden_states, w_router, gate_w, up_w, down_w,
           shared_gate_w, shared_up_w, shared_down_w):
    shape = hidden_states.shape
    x2 = hidden_states.reshape(N, D)
    gw = gate_w.astype(jnp.bfloat16)
    uw = up_w.astype(jnp.bfloat16)
    dw = down_w.astype(jnp.bfloat16)
    sg = shared_gate_w.astype(jnp.bfloat16)
    su = shared_up_w.astype(jnp.bfloat16)
    sd = shared_down_w.astype(jnp.bfloat16)

    route, topw = _router(x2, w_router)
    slots, blk = _plan_full(route.reshape(N))
    xs = _make_dispatch()(x2, slots)
    shared_out = _shared(x2, sg, su, sd)
    ys = _grouped_gemm(blk, xs, gw, uw, dw)
    out = _make_combine()(shared_out, ys, slots, topw.reshape(2 * N))
    return out.reshape(shape)


# speedup vs baseline: 1.0385x; 1.0228x over previous
import functools

import jax
import jax.numpy as jnp
from jax import lax
from jax.experimental import pallas as pl
from jax.experimental.pallas import tpu as pltpu
from jax.experimental.pallas import tpu_sc as plsc

N = 4096
D = 1024
E = 8
F = 512
FS = 1024
SCALE = 2.5

BMA = 1024
BMG = 256
LOG_BMG = 8
NP = 2 * N + E * BMG
NB = NP // BMG
NBPAD = 64


def _router_shared_body(x_ref, wr_ref, sg_ref, su_ref, sd_ref,
                        shared_ref, route_ref, topw_ref):
    x32 = x_ref[...]
    xb = x32.astype(jnp.bfloat16)
    logits = jnp.dot(x32, wr_ref[...], preferred_element_type=jnp.float32)
    m = jnp.max(logits, axis=-1, keepdims=True)
    p = jnp.exp(logits - m)
    scores = p / jnp.sum(p, axis=-1, keepdims=True)
    lane = lax.broadcasted_iota(jnp.int32, scores.shape, 1)
    v1 = jnp.max(scores, axis=-1, keepdims=True)
    i1 = jnp.min(jnp.where(scores == v1, lane, E), axis=-1, keepdims=True)
    masked = jnp.where(lane == i1, -1e30, scores)
    v2 = jnp.max(masked, axis=-1, keepdims=True)
    i2 = jnp.min(jnp.where(masked == v2, lane, E), axis=-1, keepdims=True)
    s = v1 + v2
    route_ref[...] = i1 + E * i2
    topw_ref[...] = jnp.concatenate([v1 / s, v2 / s], axis=1) * SCALE
    g = jnp.dot(xb, sg_ref[...], preferred_element_type=jnp.float32)
    u = jnp.dot(xb, su_ref[...], preferred_element_type=jnp.float32)
    h = (g * jax.nn.sigmoid(g) * u).astype(jnp.bfloat16)
    shared_ref[...] = jnp.dot(h, sd_ref[...], preferred_element_type=jnp.float32)


def _router_shared(x2, w_router, sg, su, sd):
    rb = N // BMA
    return pl.pallas_call(
        _router_shared_body,
        grid=(rb,),
        in_specs=[
            pl.BlockSpec((BMA, D), lambda i: (i, 0)),
            pl.BlockSpec((D, E), lambda i: (0, 0)),
            pl.BlockSpec((D, FS), lambda i: (0, 0)),
            pl.BlockSpec((D, FS), lambda i: (0, 0)),
            pl.BlockSpec((FS, D), lambda i: (0, 0)),
        ],
        out_specs=[
            pl.BlockSpec((BMA, D), lambda i: (i, 0)),
            pl.BlockSpec((BMA, 1), lambda i: (i, 0)),
            pl.BlockSpec((BMA, 2), lambda i: (i, 0)),
        ],
        out_shape=[
            jax.ShapeDtypeStruct((N, D), jnp.float32),
            jax.ShapeDtypeStruct((N, 1), jnp.int32),
            jax.ShapeDtypeStruct((N, 2), jnp.float32),
        ],
        compiler_params=pltpu.CompilerParams(
            dimension_semantics=("arbitrary",),
        ),
    )(x2, w_router, sg, su, sd)


_TPW = N // 16


def _stage(x, buf_v):
    z = jnp.zeros((16,), jnp.int32)
    buf_v[pl.ds(0, 16)] = z
    buf_v[pl.ds(16, 16)] = x
    buf_v[pl.ds(32, 16)] = z


def _prefix16(x, buf_v):
    for k in (1, 2, 4, 8):
        _stage(x, buf_v)
        x = x + buf_v[pl.ds(16 - k, 16)]
    return x


def _splat_last(x, buf_v):
    for k in (1, 2, 4, 8):
        _stage(x, buf_v)
        x = jnp.maximum(x, buf_v[pl.ds(16 + k, 16)])
    return x


def _splat_lane(x, e, buf_v):
    if e:
        _stage(x, buf_v)
        y = buf_v[pl.ds(16 + e, 16)]
    else:
        y = x
    y = y * _mask01(lax.iota(jnp.int32, 16) == 0)
    for k in (1, 2, 4, 8):
        _stage(y, buf_v)
        y = jnp.maximum(y, buf_v[pl.ds(16 - k, 16)])
    return y


def _mask01(pred):
    return jnp.where(pred, jnp.full((16,), 1, jnp.int32),
                     jnp.zeros((16,), jnp.int32))


def _plan1_body(route_hbm, counts_hbm, ranks_hbm,
                pk_v, eid_v, rank_v, cnt_v, buf_v):
    wid = lax.axis_index("s")
    t0 = wid * _TPW
    pltpu.sync_copy(route_hbm.at[pl.ds(t0, _TPW)], pk_v)
    nv = _TPW // 16
    for v in range(nv):
        pk = pk_v[pl.ds(v * 16, 16)]
        eid_v[pl.ds(v * 16, 16)] = pk & 7
        eid_v[pl.ds(_TPW + v * 16, 16)] = pk >> 3
    zero16 = jnp.zeros((16,), jnp.int32)
    cnt_v[...] = zero16
    lane16 = lax.iota(jnp.int32, 16)
    for e in range(E):
        def body(v, run, e=e):
            x = eid_v[pl.ds(v * 16, 16)]
            mi = _mask01(x == e)
            s = _prefix16(mi, buf_v)
            old = rank_v[pl.ds(v * 16, 16)]
            rank_v[pl.ds(v * 16, 16)] = old + mi * (run + s - 1 - old)
            return run + _splat_last(s, buf_v)
        run_f = lax.fori_loop(0, 2 * nv, body, jnp.zeros((16,), jnp.int32))
        cur = cnt_v[...]
        cnt_v[...] = cur + _mask01(lane16 == e) * (run_f - cur)
    pltpu.sync_copy(cnt_v, counts_hbm.at[wid])
    pltpu.sync_copy(rank_v.at[pl.ds(0, _TPW)],
                    ranks_hbm.at[0, pl.ds(t0, _TPW)])
    pltpu.sync_copy(rank_v.at[pl.ds(_TPW, _TPW)],
                    ranks_hbm.at[1, pl.ds(t0, _TPW)])


def _plan2_body(route_hbm, counts_hbm, ranks_hbm, slots_hbm, blk_hbm,
                pk_v, eid_v, rank_v, dest_v, cnt_v, base_v, blk_v,
                allc_v, buf_v, basesp_v, offsp_v):
    wid = lax.axis_index("s")
    t0 = wid * _TPW
    pltpu.sync_copy(route_hbm.at[pl.ds(t0, _TPW)], pk_v)
    nv = _TPW // 16
    for v in range(nv):
        pk = pk_v[pl.ds(v * 16, 16)]
        eid_v[pl.ds(v * 16, 16)] = pk & 7
        eid_v[pl.ds(_TPW + v * 16, 16)] = pk >> 3
    pltpu.sync_copy(ranks_hbm.at[0, pl.ds(t0, _TPW)],
                    rank_v.at[pl.ds(0, _TPW)])
    pltpu.sync_copy(ranks_hbm.at[1, pl.ds(t0, _TPW)],
                    rank_v.at[pl.ds(_TPW, _TPW)])
    pltpu.sync_copy(counts_hbm, allc_v)
    zero16 = jnp.zeros((16,), jnp.int32)
    lane16 = lax.iota(jnp.int32, 16)
    widv = zero16 + wid
    total = jnp.zeros((16,), jnp.int32)
    pref = jnp.zeros((16,), jnp.int32)
    for w in range(16):
        row = allc_v[w, :]
        total = total + row
        pref = pref + row * _mask01(jnp.full((16,), w, jnp.int32) < widv)
    aligned = ((total + (BMG - 1)) >> LOG_BMG) << LOG_BMG
    offi = _prefix16(aligned, buf_v)
    base = (offi - aligned) + pref
    base_v[...] = base
    for e in range(E):
        basesp_v[e, :] = _splat_lane(base, e, buf_v)
        offsp_v[e, :] = _splat_lane(offi, e, buf_v)
    def bodyc(v, _):
        sl = pl.ds(v * 16, 16)
        ev = eid_v[sl]
        d = rank_v[sl]
        for e in range(E):
            d = d + basesp_v[e, :] * _mask01(ev == e)
        dest_v[sl] = d
        return 0
    lax.fori_loop(0, 2 * nv, bodyc, 0)
    pltpu.sync_copy(dest_v.at[pl.ds(0, _TPW)], slots_hbm.at[0, pl.ds(t0, _TPW)])
    pltpu.sync_copy(dest_v.at[pl.ds(_TPW, _TPW)], slots_hbm.at[1, pl.ds(t0, _TPW)])

    @pl.when(wid == 0)
    def _():
        for v in range(NBPAD // 16):
            pos = (lax.iota(jnp.int32, 16) + v * 16) * BMG
            cnt = jnp.zeros((16,), jnp.int32)
            for e in range(E):
                cnt = cnt + _mask01(pos >= offsp_v[e, :])
            blk_v[pl.ds(v * 16, 16)] = jnp.minimum(cnt, E - 1)
        pltpu.sync_copy(blk_v, blk_hbm)


@functools.cache
def _make_plan1():
    mesh = plsc.VectorSubcoreMesh(
        core_axis_name="c", subcore_axis_name="s", num_cores=1,
        num_subcores=16)

    @functools.partial(
        pl.kernel,
        out_type=[
            jax.ShapeDtypeStruct((16, 16), jnp.int32),
            jax.ShapeDtypeStruct((2, N), jnp.int32),
        ],
        mesh=mesh,
        scratch_types=[
            pltpu.VMEM((_TPW,), jnp.int32),
            pltpu.VMEM((2 * _TPW,), jnp.int32),
            pltpu.VMEM((2 * _TPW,), jnp.int32),
            pltpu.VMEM((16,), jnp.int32),
            pltpu.VMEM((48,), jnp.int32),
        ],
    )
    def _plan1(route_hbm, counts_hbm, ranks_hbm, *rest):
        _plan1_body(route_hbm, counts_hbm, ranks_hbm, *rest)

    return _plan1


@functools.cache
def _make_plan2():
    mesh = plsc.VectorSubcoreMesh(
        core_axis_name="c", subcore_axis_name="s", num_cores=1,
        num_subcores=16)

    @functools.partial(
        pl.kernel,
        out_type=[
            jax.ShapeDtypeStruct((2, N), jnp.int32),
            jax.ShapeDtypeStruct((NBPAD,), jnp.int32),
        ],
        mesh=mesh,
        scratch_types=[
            pltpu.VMEM((_TPW,), jnp.int32),
            pltpu.VMEM((2 * _TPW,), jnp.int32),
            pltpu.VMEM((2 * _TPW,), jnp.int32),
            pltpu.VMEM((2 * _TPW,), jnp.int32),
            pltpu.VMEM((16,), jnp.int32),
            pltpu.VMEM((16,), jnp.int32),
            pltpu.VMEM((NBPAD,), jnp.int32),
            pltpu.VMEM((16, 16), jnp.int32),
            pltpu.VMEM((48,), jnp.int32),
            pltpu.VMEM((E, 16), jnp.int32),
            pltpu.VMEM((E, 16), jnp.int32),
        ],
    )
    def _plan2(route_hbm, counts_hbm, ranks_hbm, slots_hbm, blk_hbm, *rest):
        _plan2_body(route_hbm, counts_hbm, ranks_hbm, slots_hbm, blk_hbm,
                    *rest)

    return _plan2


def _plan_full(route_flat):
    counts, ranks = _make_plan1()(route_flat)
    return _make_plan2()(route_flat, counts, ranks)


_TPD = N // 32
_CHUNK = 64
_CCH = 32


def _disp_mesh():
    return plsc.VectorSubcoreMesh(
        core_axis_name="c", subcore_axis_name="s", num_cores=2,
        num_subcores=16)


@functools.cache
def _make_dispatch():
    @functools.partial(
        pl.kernel,
        out_type=jax.ShapeDtypeStruct((NP, D), jnp.float32),
        mesh=_disp_mesh(),
        scratch_types=[
            pltpu.VMEM((_CHUNK, D), jnp.float32),
            pltpu.VMEM((2, _CHUNK), jnp.int32),
            pltpu.SemaphoreType.DMA,
        ],
    )
    def _dispatch(x_hbm, slots_hbm, xs_hbm, rows_v, idx_v, sem):
        wid = lax.axis_index("s") * 2 + lax.axis_index("c")
        for c in range(_TPD // _CHUNK):
            t0 = wid * _TPD + c * _CHUNK
            pltpu.sync_copy(x_hbm.at[pl.ds(t0, _CHUNK), :], rows_v)
            pltpu.sync_copy(slots_hbm.at[0, pl.ds(t0, _CHUNK)], idx_v.at[0])
            pltpu.sync_copy(slots_hbm.at[1, pl.ds(t0, _CHUNK)], idx_v.at[1])
            cp0 = pltpu.async_copy(rows_v, xs_hbm.at[idx_v.at[0]], sem)
            cp1 = pltpu.async_copy(rows_v, xs_hbm.at[idx_v.at[1]], sem)
            cp0.wait()
            cp1.wait()

    return _dispatch


def _gemm_body(blk_ref, xs_ref, gw_ref, uw_ref, dw_ref, ys_ref):
    xb = xs_ref[...].astype(jnp.bfloat16)
    g = jnp.dot(xb, gw_ref[0], preferred_element_type=jnp.float32)
    u = jnp.dot(xb, uw_ref[0], preferred_element_type=jnp.float32)
    h = (g * jax.nn.sigmoid(g) * u).astype(jnp.bfloat16)
    ys_ref[...] = jnp.dot(h, dw_ref[0], preferred_element_type=jnp.float32)


def _grouped_gemm(blk, xs, gw, uw, dw):
    return pl.pallas_call(
        _gemm_body,
        grid_spec=pltpu.PrefetchScalarGridSpec(
            num_scalar_prefetch=1,
            grid=(NB,),
            in_specs=[
                pl.BlockSpec((BMG, D), lambda i, blk_ref: (i, 0)),
                pl.BlockSpec((1, D, F), lambda i, blk_ref: (blk_ref[i], 0, 0)),
                pl.BlockSpec((1, D, F), lambda i, blk_ref: (blk_ref[i], 0, 0)),
                pl.BlockSpec((1, F, D), lambda i, blk_ref: (blk_ref[i], 0, 0)),
            ],
            out_specs=pl.BlockSpec((BMG, D), lambda i, blk_ref: (i, 0)),
        ),
        out_shape=jax.ShapeDtypeStruct((NP, D), jnp.float32),
        compiler_params=pltpu.CompilerParams(
            dimension_semantics=("arbitrary",),
        ),
    )(blk, xs, gw, uw, dw)


@functools.cache
def _make_combine():
    @functools.partial(
        pl.kernel,
        out_type=jax.ShapeDtypeStruct((N, D), jnp.float32),
        mesh=_disp_mesh(),
        scratch_types=[
            pltpu.VMEM((_CCH, D), jnp.float32),
            pltpu.VMEM((_CCH, D), jnp.float32),
            pltpu.VMEM((_CCH, D), jnp.float32),
            pltpu.VMEM((_CCH,), jnp.int32),
            pltpu.VMEM((_CCH,), jnp.int32),
            pltpu.VMEM((2 * _CCH,), jnp.float32),
            pltpu.SemaphoreType.DMA,
        ],
    )
    def _combine(shared_hbm, ys_hbm, slots_hbm, topw_hbm, out_hbm,
                 acc_v, y0_v, y1_v, i0_v, i1_v, w_v, sem):
        wid = lax.axis_index("s") * 2 + lax.axis_index("c")
        for c in range(_TPD // _CCH):
            t0 = wid * _TPD + c * _CCH
            pltpu.sync_copy(slots_hbm.at[0, pl.ds(t0, _CCH)], i0_v)
            pltpu.sync_copy(slots_hbm.at[1, pl.ds(t0, _CCH)], i1_v)
            pltpu.sync_copy(topw_hbm.at[pl.ds(2 * t0, 2 * _CCH)], w_v)
            g0 = pltpu.async_copy(ys_hbm.at[i0_v], y0_v, sem)
            g1 = pltpu.async_copy(ys_hbm.at[i1_v], y1_v, sem)
            pltpu.sync_copy(shared_hbm.at[pl.ds(t0, _CCH), :], acc_v)
            g0.wait()
            g1.wait()
            zero16f = jnp.zeros((16,), jnp.float32)
            for g in range(_CCH // 8):
                wg = w_v[pl.ds(g * 16, 16)]
                for j in range(8):
                    r = g * 8 + j
                    w0 = zero16f + wg[2 * j]
                    w1 = zero16f + wg[2 * j + 1]
                    def bodyr(v, _, r=r, w0=w0, w1=w1):
                        sl = pl.ds(v * 16, 16)
                        acc_v[r, sl] = (acc_v[r, sl] + w0 * y0_v[r, sl]
                                        + w1 * y1_v[r, sl])
                        return 0
                    lax.fori_loop(0, D // 16, bodyr, 0)
            pltpu.sync_copy(acc_v, out_hbm.at[pl.ds(t0, _CCH), :])

    return _combine


def kernel(hidden_states, w_router, gate_w, up_w, down_w,
           shared_gate_w, shared_up_w, shared_down_w):
    shape = hidden_states.shape
    x2 = hidden_states.reshape(N, D)
    gw = gate_w.astype(jnp.bfloat16)
    uw = up_w.astype(jnp.bfloat16)
    dw = down_w.astype(jnp.bfloat16)
    sg = shared_gate_w.astype(jnp.bfloat16)
    su = shared_up_w.astype(jnp.bfloat16)
    sd = shared_down_w.astype(jnp.bfloat16)

    shared_out, route, topw = _router_shared(x2, w_router, sg, su, sd)
    slots, blk = _plan_full(route.reshape(N))
    xs = _make_dispatch()(x2, slots)
    ys = _grouped_gemm(blk, xs, gw, uw, dw)
    out = _make_combine()(shared_out, ys, slots, topw.reshape(2 * N))
    return out.reshape(shape)
